# baseline (speedup 1.0000x reference)
import functools

import jax
import jax.numpy as jnp
from jax import lax
from jax.experimental import pallas as pl
from jax.experimental.pallas import tpu as pltpu
from jax.experimental.pallas import tpu_sc as plsc


def _make_transpose(V, D, NC, NS):
    NW = NC * NS
    W = 384
    NB = 2
    FC = D // 8
    full_blocks = V // W
    tail_start = full_blocks * W
    tail_rows = V - tail_start
    base_cnt = full_blocks // NW
    rem = full_blocks % NW
    trips = base_cnt + (1 if rem else 0)
    trips += (-trips) % NB
    n_groups = (W * D) // 16
    GU = 16

    mesh = plsc.VectorSubcoreMesh(core_axis_name="c", subcore_axis_name="s")

    @functools.partial(
        pl.kernel,
        mesh=mesh,
        compiler_params=pltpu.CompilerParams(
            use_tc_tiling_on_sc=True, needs_layout_passes=False
        ),
        out_type=jax.ShapeDtypeStruct((V * D,), jnp.float32),
        scratch_types=(
            [pltpu.VMEM((D, W), jnp.float32) for _ in range(NB)]
            + [pltpu.VMEM((W * D,), jnp.float32) for _ in range(NB)]
            + [pltpu.SemaphoreType.DMA, pltpu.SemaphoreType.DMA]
        ),
    )
    def transpose(tt_hbm, tail_hbm, scratch_hbm, *refs):
        bvs = refs[:NB]
        ovs = refs[NB:2 * NB]
        rsem, wsem = refs[2 * NB], refs[2 * NB + 1]
        wid = lax.axis_index("s") * NC + lax.axis_index("c")
        cnt = base_cnt + jnp.where(wid < rem, 1, 0)
        off = wid * base_cnt + jnp.minimum(wid, rem)
        last = cnt - 1
        iota16 = lax.iota(jnp.int32, 16)
        f_vecs = [ph * 16 + iota16 for ph in range(4)]

        def blk_of(i):
            return off + jnp.minimum(i, last)

        def read(i, b):
            pltpu.async_copy(
                tt_hbm.at[:, pl.ds(blk_of(i) * W, W)], bvs[b], rsem
            )

        def absorb_read(b):
            pltpu.make_async_copy(
                tt_hbm.at[:, pl.ds(0, W)], bvs[b], rsem
            ).wait()

        for b in range(NB):
            read(b, b)

        def do_ring(p, carry):
            for b in range(NB):
                i = p * NB + b
                absorb_read(b)
                @pl.when(i >= NB)
                def _():
                    pltpu.make_async_copy(
                        ovs[b], scratch_hbm.at[pl.ds(0, W * D)], wsem
                    ).wait()

                src = bvs[b]
                dst = ovs[b]

                def tr_block(t, carry2):
                    g0 = t * GU
                    vals = []
                    for k in range(GU):
                        g = g0 + k
                        vv = lax.shift_right_logical(g, 2)
                        v_vec = iota16 * 0 + vv
                        vals.append(plsc.load_gather(src, [f_vecs[k % 4], v_vec]))
                    for k in range(GU):
                        dst[pl.ds((g0 + k) * 16, 16)] = vals[k]
                    return carry2

                lax.fori_loop(0, n_groups // GU, tr_block, 0)
                pltpu.async_copy(
                    dst, scratch_hbm.at[pl.ds(blk_of(i) * (W * D), W * D)], wsem
                )
                read(i + NB, b)
            return carry

        lax.fori_loop(0, trips // NB, do_ring, 0)
        for b in range(NB):
            absorb_read(b)
            pltpu.make_async_copy(
                ovs[b], scratch_hbm.at[pl.ds(0, W * D)], wsem
            ).wait()
        @pl.when(wid == NW - 1)
        def _():
            pltpu.sync_copy(
                tail_hbm, scratch_hbm.at[pl.ds(tail_start * D, tail_rows * D)]
            )

    return transpose, tail_start, tail_rows


def _make_gather(B, H, V, D, NC, NS):
    NW = NC * NS
    N = B * H
    b_per_w = B // NW
    n_per_w = N // NW
    C = n_per_w
    while n_per_w % C != 0 or (2 * n_per_w + 2 * C * D) > 121000:
        C = C // 2 if n_per_w % (C // 2) == 0 else C - 1
    n_chunks = n_per_w // C

    mesh = plsc.VectorSubcoreMesh(core_axis_name="c", subcore_axis_name="s")

    @functools.partial(
        pl.kernel,
        mesh=mesh,
        compiler_params=pltpu.CompilerParams(
            use_tc_tiling_on_sc=False, needs_layout_passes=False
        ),
        out_type=jax.ShapeDtypeStruct((N, D), jnp.float32),
        scratch_types=[
            pltpu.VMEM((H, b_per_w), jnp.int32),
            pltpu.VMEM((n_per_w,), jnp.int32),
            pltpu.VMEM((2, C, D), jnp.float32),
            pltpu.SemaphoreType.DMA,
            pltpu.SemaphoreType.DMA,
        ],
    )
    def emb(xt_hbm, table_hbm, out_hbm, xb_v, idx_v, rows_v, gsem, osem):
        wid = lax.axis_index("s") * NC + lax.axis_index("c")
        base = wid * n_per_w
        pltpu.sync_copy(xt_hbm.at[:, pl.ds(wid * b_per_w, b_per_w)], xb_v)

        def reorder(g, carry):
            h, bo = carry
            idx_v[pl.ds(g * 16, 16)] = plsc.load_gather(xb_v, [h, bo])
            h2 = h + 16
            wrap = h2 >= H
            h2 = jnp.where(wrap, h2 - H, h2)
            bo2 = jnp.where(wrap, bo + 1, bo)
            return (h2, bo2)

        lax.fori_loop(
            0, n_per_w // 16, reorder,
            (lax.iota(jnp.int32, 16), jnp.zeros((16,), jnp.int32)),
        )
        copies = [None, None]
        out_copies = [None, None]
        for c in range(n_chunks):
            b = c % 2
            if out_copies[b] is not None:
                out_copies[b].wait()
                out_copies[b] = None
            copies[b] = pltpu.async_copy(
                table_hbm.at[idx_v.at[pl.ds(c * C, C)]], rows_v.at[b], gsem
            )
            if c > 0:
                pb = (c - 1) % 2
                copies[pb].wait()
                out_copies[pb] = pltpu.async_copy(
                    rows_v.at[pb], out_hbm.at[pl.ds(base + (c - 1) * C, C)], osem
                )
        lb = (n_chunks - 1) % 2
        copies[lb].wait()
        if n_chunks > 1 and out_copies[(n_chunks - 2) % 2] is not None:
            out_copies[(n_chunks - 2) % 2].wait()
        pltpu.sync_copy(rows_v.at[lb], out_hbm.at[pl.ds(base + (n_chunks - 1) * C, C)])

    return emb


def kernel(x, table):
    B, H = x.shape
    V, D = table.shape
    info = plsc.get_sparse_core_info()
    NC, NS = info.num_cores, info.num_subcores
    transpose, tail_start, tail_rows = _make_transpose(V, D, NC, NS)
    emb = _make_gather(B, H, V, D, NC, NS)
    tail = lax.slice(table, (tail_start, 0), (V, D)).reshape(tail_rows * D)
    scratch = transpose(table.T, tail)
    out = emb(x.T.astype(jnp.int32), scratch.reshape(V, D))
    return out.reshape(B, H, D)


# speedup vs baseline: 1.0023x; 1.0023x over previous
import functools

import jax
import jax.numpy as jnp
from jax import lax
from jax.experimental import pallas as pl
from jax.experimental.pallas import tpu as pltpu
from jax.experimental.pallas import tpu_sc as plsc


def _make_transpose(V, D, NC, NS):
    NW = NC * NS
    W = 384
    NB = 2
    FC = D // 8
    full_blocks = V // W
    tail_start = full_blocks * W
    tail_rows = V - tail_start
    base_cnt = full_blocks // NW
    rem = full_blocks % NW
    trips = base_cnt + (1 if rem else 0)
    trips += (-trips) % NB
    n_groups = (W * D) // 16
    GU = 16

    mesh = plsc.VectorSubcoreMesh(core_axis_name="c", subcore_axis_name="s")

    @functools.partial(
        pl.kernel,
        mesh=mesh,
        compiler_params=pltpu.CompilerParams(
            use_tc_tiling_on_sc=True, needs_layout_passes=False
        ),
        out_type=jax.ShapeDtypeStruct((V * D,), jnp.float32),
        scratch_types=(
            [pltpu.VMEM((D, W + 1), jnp.float32) for _ in range(NB)]
            + [pltpu.VMEM((W * D,), jnp.float32) for _ in range(NB)]
            + [pltpu.SemaphoreType.DMA, pltpu.SemaphoreType.DMA]
        ),
    )
    def transpose(tt_hbm, tail_hbm, scratch_hbm, *refs):
        bvs = refs[:NB]
        ovs = refs[NB:2 * NB]
        rsem, wsem = refs[2 * NB], refs[2 * NB + 1]
        wid = lax.axis_index("s") * NC + lax.axis_index("c")
        cnt = base_cnt + jnp.where(wid < rem, 1, 0)
        off = wid * base_cnt + jnp.minimum(wid, rem)
        last = cnt - 1
        iota16 = lax.iota(jnp.int32, 16)
        f_vecs = [ph * 16 + iota16 for ph in range(4)]

        def blk_of(i):
            return off + jnp.minimum(i, last)

        def read(i, b):
            pltpu.async_copy(
                tt_hbm.at[:, pl.ds(blk_of(i) * W, W)], bvs[b].at[:, pl.ds(0, W)], rsem
            )

        def absorb_read(b):
            pltpu.make_async_copy(
                tt_hbm.at[:, pl.ds(0, W)], bvs[b].at[:, pl.ds(0, W)], rsem
            ).wait()

        for b in range(NB):
            read(b, b)

        def do_ring(p, carry):
            for b in range(NB):
                i = p * NB + b
                absorb_read(b)
                @pl.when(i >= NB)
                def _():
                    pltpu.make_async_copy(
                        ovs[b], scratch_hbm.at[pl.ds(0, W * D)], wsem
                    ).wait()

                src = bvs[b]
                dst = ovs[b]

                def tr_block(t, carry2):
                    g0 = t * GU
                    vals = []
                    for k in range(GU):
                        g = g0 + k
                        vv = lax.shift_right_logical(g, 2)
                        v_vec = iota16 * 0 + vv
                        vals.append(plsc.load_gather(src, [f_vecs[k % 4], v_vec]))
                    for k in range(GU):
                        dst[pl.ds((g0 + k) * 16, 16)] = vals[k]
                    return carry2

                lax.fori_loop(0, n_groups // GU, tr_block, 0)
                pltpu.async_copy(
                    dst, scratch_hbm.at[pl.ds(blk_of(i) * (W * D), W * D)], wsem
                )
                read(i + NB, b)
            return carry

        lax.fori_loop(0, trips // NB, do_ring, 0)
        for b in range(NB):
            absorb_read(b)
            pltpu.make_async_copy(
                ovs[b], scratch_hbm.at[pl.ds(0, W * D)], wsem
            ).wait()
        @pl.when(wid == NW - 1)
        def _():
            pltpu.sync_copy(
                tail_hbm, scratch_hbm.at[pl.ds(tail_start * D, tail_rows * D)]
            )

    return transpose, tail_start, tail_rows


def _make_gather(B, H, V, D, NC, NS):
    NW = NC * NS
    N = B * H
    b_per_w = B // NW
    n_per_w = N // NW
    C = n_per_w
    while n_per_w % C != 0 or (2 * n_per_w + 2 * C * D) > 121000:
        C = C // 2 if n_per_w % (C // 2) == 0 else C - 1
    n_chunks = n_per_w // C

    mesh = plsc.VectorSubcoreMesh(core_axis_name="c", subcore_axis_name="s")

    @functools.partial(
        pl.kernel,
        mesh=mesh,
        compiler_params=pltpu.CompilerParams(
            use_tc_tiling_on_sc=False, needs_layout_passes=False
        ),
        out_type=jax.ShapeDtypeStruct((N, D), jnp.float32),
        scratch_types=[
            pltpu.VMEM((H, b_per_w + 1), jnp.int32),
            pltpu.VMEM((n_per_w,), jnp.int32),
            pltpu.VMEM((2, C, D), jnp.float32),
            pltpu.SemaphoreType.DMA,
            pltpu.SemaphoreType.DMA,
        ],
    )
    def emb(xt_hbm, table_hbm, out_hbm, xb_v, idx_v, rows_v, gsem, osem):
        wid = lax.axis_index("s") * NC + lax.axis_index("c")
        base = wid * n_per_w
        pltpu.sync_copy(xt_hbm.at[:, pl.ds(wid * b_per_w, b_per_w)], xb_v.at[:, pl.ds(0, b_per_w)])

        def reorder(g, carry):
            h, bo = carry
            idx_v[pl.ds(g * 16, 16)] = plsc.load_gather(xb_v, [h, bo])
            h2 = h + 16
            wrap = h2 >= H
            h2 = jnp.where(wrap, h2 - H, h2)
            bo2 = jnp.where(wrap, bo + 1, bo)
            return (h2, bo2)

        lax.fori_loop(
            0, n_per_w // 16, reorder,
            (lax.iota(jnp.int32, 16), jnp.zeros((16,), jnp.int32)),
        )
        copies = [None, None]
        out_copies = [None, None]
        for c in range(n_chunks):
            b = c % 2
            if out_copies[b] is not None:
                out_copies[b].wait()
                out_copies[b] = None
            copies[b] = pltpu.async_copy(
                table_hbm.at[idx_v.at[pl.ds(c * C, C)]], rows_v.at[b], gsem
            )
            if c > 0:
                pb = (c - 1) % 2
                copies[pb].wait()
                out_copies[pb] = pltpu.async_copy(
                    rows_v.at[pb], out_hbm.at[pl.ds(base + (c - 1) * C, C)], osem
                )
        lb = (n_chunks - 1) % 2
        copies[lb].wait()
        if n_chunks > 1 and out_copies[(n_chunks - 2) % 2] is not None:
            out_copies[(n_chunks - 2) % 2].wait()
        pltpu.sync_copy(rows_v.at[lb], out_hbm.at[pl.ds(base + (n_chunks - 1) * C, C)])

    return emb


def kernel(x, table):
    B, H = x.shape
    V, D = table.shape
    info = plsc.get_sparse_core_info()
    NC, NS = info.num_cores, info.num_subcores
    transpose, tail_start, tail_rows = _make_transpose(V, D, NC, NS)
    emb = _make_gather(B, H, V, D, NC, NS)
    tail = lax.slice(table, (tail_start, 0), (V, D)).reshape(tail_rows * D)
    scratch = transpose(table.T, tail)
    out = emb(x.T.astype(jnp.int32), scratch.reshape(V, D))
    return out.reshape(B, H, D)


# speedup vs baseline: 1.2810x; 1.2780x over previous
import functools

import jax
import jax.numpy as jnp
from jax import lax
from jax.experimental import pallas as pl
from jax.experimental.pallas import tpu as pltpu
from jax.experimental.pallas import tpu_sc as plsc


def _make_detile(V, D, NC, NS):
    NW = NC * NS
    CR = 320
    n_chunks = V // CR
    base_cnt = n_chunks // NW
    rem = n_chunks % NW
    trips = base_cnt + (1 if rem else 0)
    if trips % 2:
        trips += 1
    row_groups = D // 16

    mesh = plsc.VectorSubcoreMesh(core_axis_name="c", subcore_axis_name="s")

    @functools.partial(
        pl.kernel,
        mesh=mesh,
        compiler_params=pltpu.CompilerParams(
            use_tc_tiling_on_sc=True, needs_layout_passes=False
        ),
        out_type=jax.ShapeDtypeStruct((V * D,), jnp.float32),
        scratch_types=[
            pltpu.VMEM((CR, D), jnp.float32),
            pltpu.VMEM((CR, D), jnp.float32),
            pltpu.VMEM((CR * D,), jnp.float32),
            pltpu.VMEM((CR * D,), jnp.float32),
            pltpu.SemaphoreType.DMA,
            pltpu.SemaphoreType.DMA,
        ],
    )
    def detile(tbl_hbm, lin_hbm, bv0, bv1, fv0, fv1, rsem, wsem):
        bvs = (bv0, bv1)
        fvs = (fv0, fv1)
        wid = lax.axis_index("s") * NC + lax.axis_index("c")
        cnt = base_cnt + jnp.where(wid < rem, 1, 0)
        off = wid * base_cnt + jnp.minimum(wid, rem)
        last = cnt - 1

        def blk_of(i):
            return off + jnp.minimum(i, last)

        def read(i, b):
            pltpu.async_copy(tbl_hbm.at[pl.ds(blk_of(i) * CR, CR), :], bvs[b], rsem)

        def absorb_read(b):
            pltpu.make_async_copy(tbl_hbm.at[pl.ds(0, CR), :], bvs[b], rsem).wait()

        def absorb_write(b):
            pltpu.make_async_copy(fvs[b], lin_hbm.at[pl.ds(0, CR * D)], wsem).wait()

        read(0, 0)
        read(1, 1)

        def do_pair(p, carry):
            for b in range(2):
                i = p * 2 + b
                absorb_read(b)
                @pl.when(i >= 2)
                def _():
                    absorb_write(b)

                src = bvs[b]
                dst = fvs[b]

                def vcopy(r, carry2):
                    for k in range(row_groups):
                        dst[pl.ds(r * D + k * 16, 16)] = src[r, pl.ds(k * 16, 16)]
                    return carry2

                lax.fori_loop(0, CR, vcopy, 0)
                pltpu.async_copy(
                    dst, lin_hbm.at[pl.ds(blk_of(i) * (CR * D), CR * D)], wsem
                )
                read(i + 2, b)
            return carry

        lax.fori_loop(0, trips // 2, do_pair, 0)
        for b in range(2):
            absorb_read(b)
            absorb_write(b)

    return detile


def _make_gather(B, H, V, D, NC, NS):
    NW = NC * NS
    N = B * H
    b_per_w = B // NW
    n_per_w = N // NW
    C = n_per_w
    while n_per_w % C != 0 or (2 * n_per_w + 2 * C * D) > 121000:
        C = C // 2 if n_per_w % (C // 2) == 0 else C - 1
    n_chunks = n_per_w // C

    mesh = plsc.VectorSubcoreMesh(core_axis_name="c", subcore_axis_name="s")

    @functools.partial(
        pl.kernel,
        mesh=mesh,
        compiler_params=pltpu.CompilerParams(
            use_tc_tiling_on_sc=False, needs_layout_passes=False
        ),
        out_type=jax.ShapeDtypeStruct((N, D), jnp.float32),
        scratch_types=[
            pltpu.VMEM((H, b_per_w + 1), jnp.int32),
            pltpu.VMEM((n_per_w,), jnp.int32),
            pltpu.VMEM((2, C, D), jnp.float32),
            pltpu.SemaphoreType.DMA,
            pltpu.SemaphoreType.DMA,
        ],
    )
    def emb(xt_hbm, table_hbm, out_hbm, xb_v, idx_v, rows_v, gsem, osem):
        wid = lax.axis_index("s") * NC + lax.axis_index("c")
        base = wid * n_per_w
        pltpu.sync_copy(
            xt_hbm.at[:, pl.ds(wid * b_per_w, b_per_w)],
            xb_v.at[:, pl.ds(0, b_per_w)],
        )

        def reorder(g, carry):
            h, bo = carry
            idx_v[pl.ds(g * 16, 16)] = plsc.load_gather(xb_v, [h, bo])
            h2 = h + 16
            wrap = h2 >= H
            h2 = jnp.where(wrap, h2 - H, h2)
            bo2 = jnp.where(wrap, bo + 1, bo)
            return (h2, bo2)

        lax.fori_loop(
            0, n_per_w // 16, reorder,
            (lax.iota(jnp.int32, 16), jnp.zeros((16,), jnp.int32)),
        )
        copies = [None, None]
        out_copies = [None, None]
        for c in range(n_chunks):
            b = c % 2
            if out_copies[b] is not None:
                out_copies[b].wait()
                out_copies[b] = None
            copies[b] = pltpu.async_copy(
                table_hbm.at[idx_v.at[pl.ds(c * C, C)]], rows_v.at[b], gsem
            )
            if c > 0:
                pb = (c - 1) % 2
                copies[pb].wait()
                out_copies[pb] = pltpu.async_copy(
                    rows_v.at[pb], out_hbm.at[pl.ds(base + (c - 1) * C, C)], osem
                )
        lb = (n_chunks - 1) % 2
        copies[lb].wait()
        if n_chunks > 1 and out_copies[(n_chunks - 2) % 2] is not None:
            out_copies[(n_chunks - 2) % 2].wait()
        pltpu.sync_copy(rows_v.at[lb], out_hbm.at[pl.ds(base + (n_chunks - 1) * C, C)])

    return emb


def kernel(x, table):
    B, H = x.shape
    V, D = table.shape
    info = plsc.get_sparse_core_info()
    detile = _make_detile(V, D, info.num_cores, info.num_subcores)
    emb = _make_gather(B, H, V, D, info.num_cores, info.num_subcores)
    lin = detile(table)
    out = emb(x.T.astype(jnp.int32), lin.reshape(V, D))
    return out.reshape(B, H, D)


# speedup vs baseline: 1.6333x; 1.2750x over previous
import functools

import jax
import jax.numpy as jnp
from jax import lax
from jax.experimental import pallas as pl
from jax.experimental.pallas import tpu as pltpu
from jax.experimental.pallas import tpu_sc as plsc


def _make_gather(B, H, V, D, NC, NS):
    NW = NC * NS
    N = B * H
    b_per_w = B // NW
    n_per_w = N // NW
    C = n_per_w
    while n_per_w % C != 0 or (2 * n_per_w + 2 * C * D) > 121000:
        C = C // 2 if n_per_w % (C // 2) == 0 else C - 1
    n_chunks = n_per_w // C

    mesh = plsc.VectorSubcoreMesh(core_axis_name="c", subcore_axis_name="s")

    @functools.partial(
        pl.kernel,
        mesh=mesh,
        compiler_params=pltpu.CompilerParams(
            use_tc_tiling_on_sc=False, needs_layout_passes=False
        ),
        out_type=jax.ShapeDtypeStruct((N, D), jnp.float32),
        scratch_types=[
            pltpu.VMEM((H, b_per_w + 1), jnp.int32),
            pltpu.VMEM((n_per_w,), jnp.int32),
            pltpu.VMEM((2, C, D), jnp.float32),
            pltpu.SemaphoreType.DMA,
            pltpu.SemaphoreType.DMA,
        ],
    )
    def emb(xt_hbm, table_hbm, out_hbm, xb_v, idx_v, rows_v, gsem, osem):
        wid = lax.axis_index("s") * NC + lax.axis_index("c")
        base = wid * n_per_w
        pltpu.sync_copy(
            xt_hbm.at[:, pl.ds(wid * b_per_w, b_per_w)],
            xb_v.at[:, pl.ds(0, b_per_w)],
        )

        def reorder(g, carry):
            h, bo = carry
            idx_v[pl.ds(g * 16, 16)] = plsc.load_gather(xb_v, [h, bo])
            h2 = h + 16
            wrap = h2 >= H
            h2 = jnp.where(wrap, h2 - H, h2)
            bo2 = jnp.where(wrap, bo + 1, bo)
            return (h2, bo2)

        lax.fori_loop(
            0, n_per_w // 16, reorder,
            (lax.iota(jnp.int32, 16), jnp.zeros((16,), jnp.int32)),
        )
        copies = [None, None]
        out_copies = [None, None]
        for c in range(n_chunks):
            b = c % 2
            if out_copies[b] is not None:
                out_copies[b].wait()
                out_copies[b] = None
            copies[b] = pltpu.async_copy(
                table_hbm.at[idx_v.at[pl.ds(c * C, C)]], rows_v.at[b], gsem
            )
            if c > 0:
                pb = (c - 1) % 2
                copies[pb].wait()
                out_copies[pb] = pltpu.async_copy(
                    rows_v.at[pb], out_hbm.at[pl.ds(base + (c - 1) * C, C)], osem
                )
        lb = (n_chunks - 1) % 2
        copies[lb].wait()
        if n_chunks > 1 and out_copies[(n_chunks - 2) % 2] is not None:
            out_copies[(n_chunks - 2) % 2].wait()
        pltpu.sync_copy(rows_v.at[lb], out_hbm.at[pl.ds(base + (n_chunks - 1) * C, C)])

    return emb


def kernel(x, table):
    B, H = x.shape
    V, D = table.shape
    info = plsc.get_sparse_core_info()
    emb = _make_gather(B, H, V, D, info.num_cores, info.num_subcores)
    out = emb(x.T.astype(jnp.int32), table)
    return out.reshape(B, H, D)
